# Initial kernel scaffold; baseline (speedup 1.0000x reference)
import functools

import jax
import jax.numpy as jnp
from jax import lax
from jax.experimental import pallas as pl
from jax.experimental.pallas import tpu as pltpu
from jax.experimental.pallas import tpu_sc as plsc

_N = 10000
_E = 320000
_D = 128
_NC = 2
_NS = 16
_NW = _NC * _NS
_EPW = _E // _NW
_C = 80
_G = _EPW // _C
_RPT = _N // _NS
_ZR = 125
_ZS = _RPT // _ZR
_BM = 1000

_f32 = jnp.float32
_mesh = plsc.VectorSubcoreMesh(core_axis_name="c", subcore_axis_name="s")


@functools.partial(
    pl.kernel,
    out_type=jax.ShapeDtypeStruct((_NC, _N, 16), _f32),
    mesh=_mesh,
    scratch_types=[
        pltpu.VMEM((_G, _C), jnp.int32),
        pltpu.VMEM((_C, 16), _f32),
        pltpu.VMEM((_ZR, 16), _f32),
        pltpu.VMEM_SHARED((_N, 16), _f32),
    ],
)
def _deg_kernel(dst_hbm, cnt_hbm, dsts, ones, zb, acc):
    cid = lax.axis_index("c")
    sid = lax.axis_index("s")
    wid = cid * _NS + sid
    ov = jnp.ones((16,), _f32)
    zv = jnp.zeros((16,), _f32)

    @pl.loop(0, _C)
    def _(i):
        ones[i] = ov

    @pl.loop(0, _ZR)
    def _(i):
        zb[i] = zv

    @pl.loop(0, _ZS)
    def _(k):
        pltpu.sync_copy(zb, acc.at[pl.ds(sid * _RPT + k * _ZR, _ZR)])

    pltpu.sync_copy(dst_hbm.at[wid], dsts)
    plsc.subcore_barrier()

    @pl.loop(0, _G)
    def _(g):
        pltpu.sync_copy(ones, acc.at[dsts.at[g]], add=True)

    plsc.subcore_barrier()

    @pl.loop(0, _ZS)
    def _(k):
        base = sid * _RPT + k * _ZR
        pltpu.sync_copy(acc.at[pl.ds(base, _ZR)], cnt_hbm.at[cid, pl.ds(base, _ZR)])


@functools.partial(
    pl.kernel,
    out_type=jax.ShapeDtypeStruct((_NC, _N, _D), _f32),
    mesh=_mesh,
    scratch_types=[
        pltpu.VMEM((_G, _C), jnp.int32),
        pltpu.VMEM((_G, _C), jnp.int32),
        pltpu.VMEM((_C, _D), _f32),
        pltpu.VMEM((_ZR, _D), _f32),
        pltpu.VMEM_SHARED((_N, _D), _f32),
        pltpu.SemaphoreType.DMA,
    ],
)
def _spmm(u_hbm, src_hbm, dst_hbm, out_hbm, srcs, dsts, rows, zb, acc, sem):
    cid = lax.axis_index("c")
    sid = lax.axis_index("s")
    wid = cid * _NS + sid
    zv = jnp.zeros((16,), _f32)

    @pl.loop(0, _ZR)
    def _(i):
        for j in range(_D // 16):
            zb[i, pl.ds(16 * j, 16)] = zv

    @pl.loop(0, _ZS)
    def _(k):
        pltpu.sync_copy(zb, acc.at[pl.ds(sid * _RPT + k * _ZR, _ZR)])

    pltpu.sync_copy(src_hbm.at[wid], srcs)
    pltpu.sync_copy(dst_hbm.at[wid], dsts)
    plsc.subcore_barrier()

    @pl.loop(0, _G)
    def _(g):
        pltpu.async_copy(u_hbm.at[srcs.at[g]], rows, sem).wait()
        pltpu.sync_copy(rows, acc.at[dsts.at[g]], add=True)

    plsc.subcore_barrier()

    @pl.loop(0, _ZS)
    def _(k):
        base = sid * _RPT + k * _ZR
        pltpu.sync_copy(acc.at[pl.ds(base, _ZR)], out_hbm.at[cid, pl.ds(base, _ZR)])


def _tc_first_body(x_ref, w_ref, cnt_ref, u_ref, dinv_ref):
    deg = cnt_ref[0, :, 0:1] + cnt_ref[1, :, 0:1] + 1.0
    dinv = lax.rsqrt(deg)
    dinv_ref[...] = dinv
    u_ref[...] = jnp.dot(x_ref[...], w_ref[...], preferred_element_type=_f32) * dinv


_tc_first = pl.pallas_call(
    _tc_first_body,
    grid=(_N // _BM,),
    in_specs=[
        pl.BlockSpec((_BM, _D), lambda i: (i, 0)),
        pl.BlockSpec((_D, _D), lambda i: (0, 0)),
        pl.BlockSpec((2, _BM, 16), lambda i: (0, i, 0)),
    ],
    out_specs=[
        pl.BlockSpec((_BM, _D), lambda i: (i, 0)),
        pl.BlockSpec((_BM, 1), lambda i: (i, 0)),
    ],
    out_shape=[
        jax.ShapeDtypeStruct((_N, _D), _f32),
        jax.ShapeDtypeStruct((_N, 1), _f32),
    ],
)


def _tc_mid_body(s_ref, u_ref, dinv_ref, b_ref, w_ref, o_ref):
    dinv = dinv_ref[...]
    agg = s_ref[0] + s_ref[1] + u_ref[...]
    h = jnp.maximum(agg * dinv + b_ref[...], 0.0)
    o_ref[...] = jnp.dot(h, w_ref[...], preferred_element_type=_f32) * dinv


_tc_mid = pl.pallas_call(
    _tc_mid_body,
    grid=(_N // _BM,),
    in_specs=[
        pl.BlockSpec((2, _BM, _D), lambda i: (0, i, 0)),
        pl.BlockSpec((_BM, _D), lambda i: (i, 0)),
        pl.BlockSpec((_BM, 1), lambda i: (i, 0)),
        pl.BlockSpec((1, _D), lambda i: (0, 0)),
        pl.BlockSpec((_D, _D), lambda i: (0, 0)),
    ],
    out_specs=pl.BlockSpec((_BM, _D), lambda i: (i, 0)),
    out_shape=jax.ShapeDtypeStruct((_N, _D), _f32),
)


def _tc_last_body(s_ref, u_ref, dinv_ref, b_ref, wl_ref, bl_ref, o_ref):
    agg = s_ref[0] + s_ref[1] + u_ref[...]
    h = jnp.maximum(agg * dinv_ref[...] + b_ref[...], 0.0)
    o_ref[...] = jnp.dot(h, wl_ref[...], preferred_element_type=_f32) + bl_ref[...]


_tc_last = pl.pallas_call(
    _tc_last_body,
    grid=(_N // _BM,),
    in_specs=[
        pl.BlockSpec((2, _BM, _D), lambda i: (0, i, 0)),
        pl.BlockSpec((_BM, _D), lambda i: (i, 0)),
        pl.BlockSpec((_BM, 1), lambda i: (i, 0)),
        pl.BlockSpec((1, _D), lambda i: (0, 0)),
        pl.BlockSpec((_D, 1), lambda i: (0, 0)),
        pl.BlockSpec((1, 1), lambda i: (0, 0)),
    ],
    out_specs=pl.BlockSpec((_BM, 1), lambda i: (i, 0)),
    out_shape=jax.ShapeDtypeStruct((_N, 1), _f32),
)


def kernel(x, edge_index, W0, b0, W1, b1, W2, b2, Wl, bl):
    src = edge_index[0].reshape(_NW, _G, _C)
    dst = edge_index[1].reshape(_NW, _G, _C)
    cnt = _deg_kernel(dst)
    u0, dinv = _tc_first(x, W0, cnt)
    s0 = _spmm(u0, src, dst)
    u1 = _tc_mid(s0, u0, dinv, b0.reshape(1, _D), W1)
    s1 = _spmm(u1, src, dst)
    u2 = _tc_mid(s1, u1, dinv, b1.reshape(1, _D), W2)
    s2 = _spmm(u2, src, dst)
    out = _tc_last(s2, u2, dinv, b2.reshape(1, _D), Wl, bl.reshape(1, 1))
    return out[:, 0]


# speedup vs baseline: 16.7944x; 16.7944x over previous
import functools

import jax
import jax.numpy as jnp
from jax import lax
from jax.experimental import pallas as pl
from jax.experimental.pallas import tpu as pltpu
from jax.experimental.pallas import tpu_sc as plsc

_N = 10000
_E = 320000
_D = 128
_NC = 2
_NS = 16
_NW = _NC * _NS
_EPW = _E // _NW
_C = 80
_G = _EPW // _C
_ZCH = 80
_NZC = _N // _ZCH
_BM = 1000

_f32 = jnp.float32


def _deg_body(dst_hbm, cnt_hbm, dsts, ones, zb, acc):
    cid = lax.axis_index("c")
    sid = lax.axis_index("s")
    wid = cid * _NS + sid
    ov = jnp.ones((16,), _f32)
    zv = jnp.zeros((16,), _f32)

    @pl.loop(0, _C)
    def _(i):
        ones[i] = ov

    @pl.loop(0, _ZCH)
    def _(i):
        zb[i] = zv

    @pl.loop(sid, _NZC, step=_NS)
    def _(c):
        pltpu.sync_copy(zb, acc.at[pl.ds(c * _ZCH, _ZCH)])

    pltpu.sync_copy(dst_hbm.at[wid], dsts)
    plsc.subcore_barrier()

    @pl.loop(0, _G)
    def _(g):
        pltpu.sync_copy(ones, acc.at[dsts.at[g]], add=True)

    plsc.subcore_barrier()

    @pl.loop(sid, _NZC, step=_NS)
    def _(c):
        base = c * _ZCH
        pltpu.sync_copy(acc.at[pl.ds(base, _ZCH)], cnt_hbm.at[cid, pl.ds(base, _ZCH)])


def _spmm_body(u_hbm, src_hbm, dst_hbm, out_hbm, srcs, dsts, rows, acc, sem):
    cid = lax.axis_index("c")
    sid = lax.axis_index("s")
    wid = cid * _NS + sid
    zv = jnp.zeros((16,), _f32)

    @pl.loop(0, _ZCH)
    def _(i):
        for j in range(_D // 16):
            rows[i, pl.ds(16 * j, 16)] = zv

    @pl.loop(sid, _NZC, step=_NS)
    def _(c):
        pltpu.sync_copy(rows, acc.at[pl.ds(c * _ZCH, _ZCH)])

    pltpu.sync_copy(src_hbm.at[wid], srcs)
    pltpu.sync_copy(dst_hbm.at[wid], dsts)
    plsc.subcore_barrier()

    @pl.loop(0, _G)
    def _(g):
        pltpu.async_copy(u_hbm.at[srcs.at[g]], rows, sem).wait()
        pltpu.sync_copy(rows, acc.at[dsts.at[g]], add=True)

    plsc.subcore_barrier()

    @pl.loop(sid, _NZC, step=_NS)
    def _(c):
        base = c * _ZCH
        pltpu.sync_copy(acc.at[pl.ds(base, _ZCH)], out_hbm.at[cid, pl.ds(base, _ZCH)])


@functools.cache
def _sc_kernels():
    mesh = plsc.VectorSubcoreMesh(
        core_axis_name="c", subcore_axis_name="s", num_cores=_NC, num_subcores=_NS
    )
    deg = pl.kernel(
        _deg_body,
        out_type=jax.ShapeDtypeStruct((_NC, _N, 16), _f32),
        mesh=mesh,
        scratch_types=[
            pltpu.VMEM((_G, _C), jnp.int32),
            pltpu.VMEM((_C, 16), _f32),
            pltpu.VMEM((_ZCH, 16), _f32),
            pltpu.VMEM_SHARED((_N, 16), _f32),
        ],
    )
    spmm = pl.kernel(
        _spmm_body,
        out_type=jax.ShapeDtypeStruct((_NC, _N, _D), _f32),
        mesh=mesh,
        scratch_types=[
            pltpu.VMEM((_G, _C), jnp.int32),
            pltpu.VMEM((_G, _C), jnp.int32),
            pltpu.VMEM((_C, _D), _f32),
            pltpu.VMEM_SHARED((_N, _D), _f32),
            pltpu.SemaphoreType.DMA,
        ],
    )
    return deg, spmm


def _tc_first_body(x_ref, w_ref, cnt_ref, u_ref, dinv_ref):
    deg = cnt_ref[0, :, 0:1] + cnt_ref[1, :, 0:1] + 1.0
    dinv = lax.rsqrt(deg)
    dinv_ref[...] = dinv
    u_ref[...] = jnp.dot(x_ref[...], w_ref[...], preferred_element_type=_f32) * dinv


_tc_first = pl.pallas_call(
    _tc_first_body,
    grid=(_N // _BM,),
    in_specs=[
        pl.BlockSpec((_BM, _D), lambda i: (i, 0)),
        pl.BlockSpec((_D, _D), lambda i: (0, 0)),
        pl.BlockSpec((2, _BM, 16), lambda i: (0, i, 0)),
    ],
    out_specs=[
        pl.BlockSpec((_BM, _D), lambda i: (i, 0)),
        pl.BlockSpec((_BM, 1), lambda i: (i, 0)),
    ],
    out_shape=[
        jax.ShapeDtypeStruct((_N, _D), _f32),
        jax.ShapeDtypeStruct((_N, 1), _f32),
    ],
)


def _tc_mid_body(s_ref, u_ref, dinv_ref, b_ref, w_ref, o_ref):
    dinv = dinv_ref[...]
    agg = s_ref[0] + s_ref[1] + u_ref[...]
    h = jnp.maximum(agg * dinv + b_ref[...], 0.0)
    o_ref[...] = jnp.dot(h, w_ref[...], preferred_element_type=_f32) * dinv


_tc_mid = pl.pallas_call(
    _tc_mid_body,
    grid=(_N // _BM,),
    in_specs=[
        pl.BlockSpec((2, _BM, _D), lambda i: (0, i, 0)),
        pl.BlockSpec((_BM, _D), lambda i: (i, 0)),
        pl.BlockSpec((_BM, 1), lambda i: (i, 0)),
        pl.BlockSpec((1, _D), lambda i: (0, 0)),
        pl.BlockSpec((_D, _D), lambda i: (0, 0)),
    ],
    out_specs=pl.BlockSpec((_BM, _D), lambda i: (i, 0)),
    out_shape=jax.ShapeDtypeStruct((_N, _D), _f32),
)


def _tc_last_body(s_ref, u_ref, dinv_ref, b_ref, wl_ref, bl_ref, o_ref):
    agg = s_ref[0] + s_ref[1] + u_ref[...]
    h = jnp.maximum(agg * dinv_ref[...] + b_ref[...], 0.0)
    o_ref[...] = jnp.dot(h, wl_ref[...], preferred_element_type=_f32) + bl_ref[...]


_tc_last = pl.pallas_call(
    _tc_last_body,
    grid=(_N // _BM,),
    in_specs=[
        pl.BlockSpec((2, _BM, _D), lambda i: (0, i, 0)),
        pl.BlockSpec((_BM, _D), lambda i: (i, 0)),
        pl.BlockSpec((_BM, 1), lambda i: (i, 0)),
        pl.BlockSpec((1, _D), lambda i: (0, 0)),
        pl.BlockSpec((_D, 1), lambda i: (0, 0)),
        pl.BlockSpec((1, 1), lambda i: (0, 0)),
    ],
    out_specs=pl.BlockSpec((_BM, 1), lambda i: (i, 0)),
    out_shape=jax.ShapeDtypeStruct((_N, 1), _f32),
)


def kernel(x, edge_index, W0, b0, W1, b1, W2, b2, Wl, bl):
    _deg_kernel, _spmm = _sc_kernels()
    src = edge_index[0].reshape(_NW, _G, _C)
    dst = edge_index[1].reshape(_NW, _G, _C)
    cnt = _deg_kernel(dst)
    u0, dinv = _tc_first(x, W0, cnt)
    s0 = _spmm(u0, src, dst)
    u1 = _tc_mid(s0, u0, dinv, b0.reshape(1, _D), W1)
    s1 = _spmm(u1, src, dst)
    u2 = _tc_mid(s1, u1, dinv, b1.reshape(1, _D), W2)
    s2 = _spmm(u2, src, dst)
    out = _tc_last(s2, u2, dinv, b2.reshape(1, _D), Wl, bl.reshape(1, 1))
    return out[:, 0]


# speedup vs baseline: 24.1692x; 1.4391x over previous
import functools

import jax
import jax.numpy as jnp
from jax import lax
from jax.experimental import pallas as pl
from jax.experimental.pallas import tpu as pltpu
from jax.experimental.pallas import tpu_sc as plsc

_N = 10000
_E = 320000
_D = 128
_NC = 2
_NS = 16
_NW = _NC * _NS
_EPW = _E // _NW
_CD = 80
_GD = _EPW // _CD
_C = 100
_G = _EPW // _C
_ZCH = 80
_NZC = _N // _ZCH
_BM = 1000

_f32 = jnp.float32


def _deg_body(dst_hbm, cnt_hbm, dsts, ones, zb, acc):
    cid = lax.axis_index("c")
    sid = lax.axis_index("s")
    wid = cid * _NS + sid
    ov = jnp.ones((16,), _f32)
    zv = jnp.zeros((16,), _f32)

    @pl.loop(0, _CD)
    def _(i):
        ones[i] = ov

    @pl.loop(0, _ZCH)
    def _(i):
        zb[i] = zv

    @pl.loop(sid, _NZC, step=_NS)
    def _(c):
        pltpu.sync_copy(zb, acc.at[pl.ds(c * _ZCH, _ZCH)])

    pltpu.sync_copy(dst_hbm.at[wid], dsts)
    plsc.subcore_barrier()

    @pl.loop(0, _GD)
    def _(g):
        pltpu.sync_copy(ones, acc.at[dsts.at[g]], add=True)

    plsc.subcore_barrier()

    @pl.loop(sid, _NZC, step=_NS)
    def _(c):
        base = c * _ZCH
        pltpu.sync_copy(acc.at[pl.ds(base, _ZCH)], cnt_hbm.at[cid, pl.ds(base, _ZCH)])


def _spmm_body(u_hbm, ei_hbm, out_hbm, idx, rows, acc, isem, gsem):
    cid = lax.axis_index("c")
    sid = lax.axis_index("s")
    wid = cid * _NS + sid
    zv = jnp.zeros((16,), _f32)

    @pl.loop(0, _ZCH)
    def _(i):
        for j in range(_D // 16):
            rows[0, i, pl.ds(16 * j, 16)] = zv

    @pl.loop(sid, _NZC, step=_NS)
    def _(c):
        pltpu.sync_copy(rows.at[0, pl.ds(0, _ZCH)], acc.at[pl.ds(c * _ZCH, _ZCH)])

    plsc.subcore_barrier()

    pltpu.async_copy(ei_hbm.at[wid, 0], idx.at[0], isem)
    pltpu.make_async_copy(ei_hbm.at[wid, 0], idx.at[0], isem).wait()
    pltpu.async_copy(ei_hbm.at[wid, 1], idx.at[1], isem)
    pltpu.async_copy(u_hbm.at[idx.at[0, 0]], rows.at[0], gsem)

    @pl.loop(0, _G - 1)
    def _(g):
        sl = lax.rem(g, 2)
        nsl = 1 - sl
        pltpu.make_async_copy(ei_hbm.at[wid, g + 1], idx.at[nsl], isem).wait()
        pltpu.async_copy(u_hbm.at[idx.at[nsl, 0]], rows.at[nsl], gsem)
        pltpu.make_async_copy(u_hbm.at[idx.at[sl, 0]], rows.at[sl], gsem).wait()
        pltpu.sync_copy(rows.at[sl], acc.at[idx.at[sl, 1]], add=True)

        @pl.when(g < _G - 2)
        def _():
            pltpu.async_copy(ei_hbm.at[wid, g + 2], idx.at[sl], isem)

    lsl = (_G - 1) % 2
    pltpu.make_async_copy(u_hbm.at[idx.at[lsl, 0]], rows.at[lsl], gsem).wait()
    pltpu.sync_copy(rows.at[lsl], acc.at[idx.at[lsl, 1]], add=True)

    plsc.subcore_barrier()

    @pl.loop(sid, _NZC, step=_NS)
    def _(c):
        base = c * _ZCH
        pltpu.sync_copy(acc.at[pl.ds(base, _ZCH)], out_hbm.at[cid, pl.ds(base, _ZCH)])


@functools.cache
def _sc_kernels():
    mesh = plsc.VectorSubcoreMesh(
        core_axis_name="c", subcore_axis_name="s", num_cores=_NC, num_subcores=_NS
    )
    deg = pl.kernel(
        _deg_body,
        out_type=jax.ShapeDtypeStruct((_NC, _N, 16), _f32),
        mesh=mesh,
        scratch_types=[
            pltpu.VMEM((_GD, _CD), jnp.int32),
            pltpu.VMEM((_CD, 16), _f32),
            pltpu.VMEM((_ZCH, 16), _f32),
            pltpu.VMEM_SHARED((_N, 16), _f32),
        ],
    )
    spmm = pl.kernel(
        _spmm_body,
        out_type=jax.ShapeDtypeStruct((_NC, _N, _D), _f32),
        mesh=mesh,
        scratch_types=[
            pltpu.VMEM((2, 2, _C), jnp.int32),
            pltpu.VMEM((2, _C, _D), _f32),
            pltpu.VMEM_SHARED((_N, _D), _f32),
            pltpu.SemaphoreType.DMA,
            pltpu.SemaphoreType.DMA,
        ],
    )
    return deg, spmm


def _tc_first_body(x_ref, w_ref, cnt_ref, u_ref, dinv_ref):
    deg = cnt_ref[0, :, 0:1] + cnt_ref[1, :, 0:1] + 1.0
    dinv = lax.rsqrt(deg)
    dinv_ref[...] = dinv
    u_ref[...] = jnp.dot(x_ref[...], w_ref[...], preferred_element_type=_f32) * dinv


_tc_first = pl.pallas_call(
    _tc_first_body,
    grid=(_N // _BM,),
    in_specs=[
        pl.BlockSpec((_BM, _D), lambda i: (i, 0)),
        pl.BlockSpec((_D, _D), lambda i: (0, 0)),
        pl.BlockSpec((2, _BM, 16), lambda i: (0, i, 0)),
    ],
    out_specs=[
        pl.BlockSpec((_BM, _D), lambda i: (i, 0)),
        pl.BlockSpec((_BM, 1), lambda i: (i, 0)),
    ],
    out_shape=[
        jax.ShapeDtypeStruct((_N, _D), _f32),
        jax.ShapeDtypeStruct((_N, 1), _f32),
    ],
)


def _tc_mid_body(s_ref, u_ref, dinv_ref, b_ref, w_ref, o_ref):
    dinv = dinv_ref[...]
    agg = s_ref[0] + s_ref[1] + u_ref[...]
    h = jnp.maximum(agg * dinv + b_ref[...], 0.0)
    o_ref[...] = jnp.dot(h, w_ref[...], preferred_element_type=_f32) * dinv


_tc_mid = pl.pallas_call(
    _tc_mid_body,
    grid=(_N // _BM,),
    in_specs=[
        pl.BlockSpec((2, _BM, _D), lambda i: (0, i, 0)),
        pl.BlockSpec((_BM, _D), lambda i: (i, 0)),
        pl.BlockSpec((_BM, 1), lambda i: (i, 0)),
        pl.BlockSpec((1, _D), lambda i: (0, 0)),
        pl.BlockSpec((_D, _D), lambda i: (0, 0)),
    ],
    out_specs=pl.BlockSpec((_BM, _D), lambda i: (i, 0)),
    out_shape=jax.ShapeDtypeStruct((_N, _D), _f32),
)


def _tc_last_body(s_ref, u_ref, dinv_ref, b_ref, wl_ref, bl_ref, o_ref):
    agg = s_ref[0] + s_ref[1] + u_ref[...]
    h = jnp.maximum(agg * dinv_ref[...] + b_ref[...], 0.0)
    o_ref[...] = jnp.dot(h, wl_ref[...], preferred_element_type=_f32) + bl_ref[...]


_tc_last = pl.pallas_call(
    _tc_last_body,
    grid=(_N // _BM,),
    in_specs=[
        pl.BlockSpec((2, _BM, _D), lambda i: (0, i, 0)),
        pl.BlockSpec((_BM, _D), lambda i: (i, 0)),
        pl.BlockSpec((_BM, 1), lambda i: (i, 0)),
        pl.BlockSpec((1, _D), lambda i: (0, 0)),
        pl.BlockSpec((_D, 1), lambda i: (0, 0)),
        pl.BlockSpec((1, 1), lambda i: (0, 0)),
    ],
    out_specs=pl.BlockSpec((_BM, 1), lambda i: (i, 0)),
    out_shape=jax.ShapeDtypeStruct((_N, 1), _f32),
)


def kernel(x, edge_index, W0, b0, W1, b1, W2, b2, Wl, bl):
    _deg_kernel, _spmm = _sc_kernels()
    dst_deg = edge_index[1].reshape(_NW, _GD, _CD)
    ei_sp = edge_index.reshape(2, _NW, _G, _C).transpose(1, 2, 0, 3)
    cnt = _deg_kernel(dst_deg)
    u0, dinv = _tc_first(x, W0, cnt)
    s0 = _spmm(u0, ei_sp)
    u1 = _tc_mid(s0, u0, dinv, b0.reshape(1, _D), W1)
    s1 = _spmm(u1, ei_sp)
    u2 = _tc_mid(s1, u1, dinv, b1.reshape(1, _D), W2)
    s2 = _spmm(u2, ei_sp)
    out = _tc_last(s2, u2, dinv, b2.reshape(1, _D), Wl, bl.reshape(1, 1))
    return out[:, 0]


# speedup vs baseline: 27.4822x; 1.1371x over previous
import functools

import jax
import jax.numpy as jnp
from jax import lax
from jax.experimental import pallas as pl
from jax.experimental.pallas import tpu as pltpu
from jax.experimental.pallas import tpu_sc as plsc

_N = 10000
_E = 320000
_D = 128
_NC = 2
_NS = 16
_NW = _NC * _NS
_EPW = _E // _NW
_CD = 80
_GD = _EPW // _CD
_C = 100
_G = _EPW // _C
_ZCH = 80
_NZC = _N // _ZCH
_BM = 1000

_f32 = jnp.float32


def _deg_body(dst_hbm, cnt_hbm, dsts, ones, zb, acc):
    cid = lax.axis_index("c")
    sid = lax.axis_index("s")
    wid = cid * _NS + sid
    ov = jnp.ones((16,), _f32)
    zv = jnp.zeros((16,), _f32)

    @pl.loop(0, _CD)
    def _(i):
        ones[i] = ov

    @pl.loop(0, _ZCH)
    def _(i):
        zb[i] = zv

    @pl.loop(sid, _NZC, step=_NS)
    def _(c):
        pltpu.sync_copy(zb, acc.at[pl.ds(c * _ZCH, _ZCH)])

    pltpu.sync_copy(dst_hbm.at[wid], dsts)
    plsc.subcore_barrier()

    @pl.loop(0, _GD)
    def _(g):
        pltpu.sync_copy(ones, acc.at[dsts.at[g]], add=True)

    plsc.subcore_barrier()

    @pl.loop(sid, _NZC, step=_NS)
    def _(c):
        base = c * _ZCH
        pltpu.sync_copy(acc.at[pl.ds(base, _ZCH)], cnt_hbm.at[cid, pl.ds(base, _ZCH)])


def _spmm_body(u_hbm, ei_hbm, out_hbm, idx, rows, acc, isem, gsem, ssem):
    cid = lax.axis_index("c")
    sid = lax.axis_index("s")
    wid = cid * _NS + sid
    zv = jnp.zeros((16,), _f32)

    @pl.loop(0, _ZCH)
    def _(i):
        for j in range(_D // 16):
            rows[0, i, pl.ds(16 * j, 16)] = zv

    @pl.loop(sid, _NZC, step=_NS)
    def _(c):
        pltpu.async_copy(rows.at[0, pl.ds(0, _ZCH)], acc.at[pl.ds(c * _ZCH, _ZCH)], gsem)

    @pl.loop(sid, _NZC, step=_NS)
    def _(c):
        pltpu.make_async_copy(rows.at[0, pl.ds(0, _ZCH)], acc.at[pl.ds(c * _ZCH, _ZCH)], gsem).wait()

    plsc.subcore_barrier()

    pltpu.async_copy(ei_hbm.at[wid, 0], idx.at[0], isem)
    pltpu.make_async_copy(ei_hbm.at[wid, 0], idx.at[0], isem).wait()
    pltpu.async_copy(ei_hbm.at[wid, 1], idx.at[1], isem)
    pltpu.async_copy(u_hbm.at[idx.at[0, 0]], rows.at[0], gsem)

    @pl.loop(0, _G - 1)
    def _(g):
        rsl = lax.rem(g, 2)
        rnsl = 1 - rsl
        isl = lax.rem(g, 3)
        inx = lax.rem(g + 1, 3)
        inx2 = lax.rem(g + 2, 3)
        pltpu.make_async_copy(ei_hbm.at[wid, g + 1], idx.at[inx], isem).wait()

        @pl.when(g >= 1)
        def _():
            pltpu.make_async_copy(rows.at[rnsl], acc.at[idx.at[inx2, 1]], ssem).wait()

        pltpu.async_copy(u_hbm.at[idx.at[inx, 0]], rows.at[rnsl], gsem)
        pltpu.make_async_copy(u_hbm.at[idx.at[isl, 0]], rows.at[rsl], gsem).wait()
        pltpu.async_copy(rows.at[rsl], acc.at[idx.at[isl, 1]], ssem, add=True)

        @pl.when(g < _G - 2)
        def _():
            pltpu.async_copy(ei_hbm.at[wid, g + 2], idx.at[inx2], isem)

    lsl = (_G - 1) % 2
    lisl = (_G - 1) % 3
    pltpu.make_async_copy(rows.at[1 - lsl], acc.at[idx.at[0, 1]], ssem).wait()
    pltpu.make_async_copy(u_hbm.at[idx.at[lisl, 0]], rows.at[lsl], gsem).wait()
    pltpu.sync_copy(rows.at[lsl], acc.at[idx.at[lisl, 1]], add=True)

    plsc.subcore_barrier()

    @pl.loop(sid, _NZC, step=_NS)
    def _(c):
        base = c * _ZCH
        pltpu.async_copy(acc.at[pl.ds(base, _ZCH)], out_hbm.at[cid, pl.ds(base, _ZCH)], gsem)

    @pl.loop(sid, _NZC, step=_NS)
    def _(c):
        base = c * _ZCH
        pltpu.make_async_copy(acc.at[pl.ds(base, _ZCH)], out_hbm.at[cid, pl.ds(base, _ZCH)], gsem).wait()


@functools.cache
def _sc_kernels():
    mesh = plsc.VectorSubcoreMesh(
        core_axis_name="c", subcore_axis_name="s", num_cores=_NC, num_subcores=_NS
    )
    deg = pl.kernel(
        _deg_body,
        out_type=jax.ShapeDtypeStruct((_NC, _N, 16), _f32),
        mesh=mesh,
        scratch_types=[
            pltpu.VMEM((_GD, _CD), jnp.int32),
            pltpu.VMEM((_CD, 16), _f32),
            pltpu.VMEM((_ZCH, 16), _f32),
            pltpu.VMEM_SHARED((_N, 16), _f32),
        ],
    )
    spmm = pl.kernel(
        _spmm_body,
        out_type=jax.ShapeDtypeStruct((_NC, _N, _D), _f32),
        mesh=mesh,
        scratch_types=[
            pltpu.VMEM((3, 2, _C), jnp.int32),
            pltpu.VMEM((2, _C, _D), _f32),
            pltpu.VMEM_SHARED((_N, _D), _f32),
            pltpu.SemaphoreType.DMA,
            pltpu.SemaphoreType.DMA,
            pltpu.SemaphoreType.DMA,
        ],
    )
    return deg, spmm


def _tc_first_body(x_ref, w_ref, cnt_ref, u_ref, dinv_ref):
    deg = cnt_ref[0, :, 0:1] + cnt_ref[1, :, 0:1] + 1.0
    dinv = lax.rsqrt(deg)
    dinv_ref[...] = dinv
    u_ref[...] = jnp.dot(x_ref[...], w_ref[...], preferred_element_type=_f32) * dinv


_tc_first = pl.pallas_call(
    _tc_first_body,
    grid=(_N // _BM,),
    in_specs=[
        pl.BlockSpec((_BM, _D), lambda i: (i, 0)),
        pl.BlockSpec((_D, _D), lambda i: (0, 0)),
        pl.BlockSpec((2, _BM, 16), lambda i: (0, i, 0)),
    ],
    out_specs=[
        pl.BlockSpec((_BM, _D), lambda i: (i, 0)),
        pl.BlockSpec((_BM, 1), lambda i: (i, 0)),
    ],
    out_shape=[
        jax.ShapeDtypeStruct((_N, _D), _f32),
        jax.ShapeDtypeStruct((_N, 1), _f32),
    ],
)


def _tc_mid_body(s_ref, u_ref, dinv_ref, b_ref, w_ref, o_ref):
    dinv = dinv_ref[...]
    agg = s_ref[0] + s_ref[1] + u_ref[...]
    h = jnp.maximum(agg * dinv + b_ref[...], 0.0)
    o_ref[...] = jnp.dot(h, w_ref[...], preferred_element_type=_f32) * dinv


_tc_mid = pl.pallas_call(
    _tc_mid_body,
    grid=(_N // _BM,),
    in_specs=[
        pl.BlockSpec((2, _BM, _D), lambda i: (0, i, 0)),
        pl.BlockSpec((_BM, _D), lambda i: (i, 0)),
        pl.BlockSpec((_BM, 1), lambda i: (i, 0)),
        pl.BlockSpec((1, _D), lambda i: (0, 0)),
        pl.BlockSpec((_D, _D), lambda i: (0, 0)),
    ],
    out_specs=pl.BlockSpec((_BM, _D), lambda i: (i, 0)),
    out_shape=jax.ShapeDtypeStruct((_N, _D), _f32),
)


def _tc_last_body(s_ref, u_ref, dinv_ref, b_ref, wl_ref, bl_ref, o_ref):
    agg = s_ref[0] + s_ref[1] + u_ref[...]
    h = jnp.maximum(agg * dinv_ref[...] + b_ref[...], 0.0)
    o_ref[...] = jnp.dot(h, wl_ref[...], preferred_element_type=_f32) + bl_ref[...]


_tc_last = pl.pallas_call(
    _tc_last_body,
    grid=(_N // _BM,),
    in_specs=[
        pl.BlockSpec((2, _BM, _D), lambda i: (0, i, 0)),
        pl.BlockSpec((_BM, _D), lambda i: (i, 0)),
        pl.BlockSpec((_BM, 1), lambda i: (i, 0)),
        pl.BlockSpec((1, _D), lambda i: (0, 0)),
        pl.BlockSpec((_D, 1), lambda i: (0, 0)),
        pl.BlockSpec((1, 1), lambda i: (0, 0)),
    ],
    out_specs=pl.BlockSpec((_BM, 1), lambda i: (i, 0)),
    out_shape=jax.ShapeDtypeStruct((_N, 1), _f32),
)


def kernel(x, edge_index, W0, b0, W1, b1, W2, b2, Wl, bl):
    _deg_kernel, _spmm = _sc_kernels()
    dst_deg = edge_index[1].reshape(_NW, _GD, _CD)
    ei_sp = edge_index.reshape(2, _NW, _G, _C).transpose(1, 2, 0, 3)
    cnt = _deg_kernel(dst_deg)
    u0, dinv = _tc_first(x, W0, cnt)
    s0 = _spmm(u0, ei_sp)
    u1 = _tc_mid(s0, u0, dinv, b0.reshape(1, _D), W1)
    s1 = _spmm(u1, ei_sp)
    u2 = _tc_mid(s1, u1, dinv, b1.reshape(1, _D), W2)
    s2 = _spmm(u2, ei_sp)
    out = _tc_last(s2, u2, dinv, b2.reshape(1, _D), Wl, bl.reshape(1, 1))
    return out[:, 0]


# speedup vs baseline: 28.0190x; 1.0195x over previous
import functools

import jax
import jax.numpy as jnp
from jax import lax
from jax.experimental import pallas as pl
from jax.experimental.pallas import tpu as pltpu
from jax.experimental.pallas import tpu_sc as plsc

_N = 10000
_E = 320000
_D = 128
_NC = 2
_NS = 16
_NW = _NC * _NS
_EPW = _E // _NW
_CD = 80
_GD = _EPW // _CD
_C = 100
_G = _EPW // _C
_ZCH = 80
_NZC = _N // _ZCH
_BM = 1000

_f32 = jnp.float32


def _deg_body(dst_hbm, cnt_hbm, dsts, ones, zb, acc, isem, gsem, ssem):
    cid = lax.axis_index("c")
    sid = lax.axis_index("s")
    wid = cid * _NS + sid
    ov = jnp.ones((16,), _f32)
    zv = jnp.zeros((16,), _f32)

    pltpu.async_copy(dst_hbm.at[wid], dsts, isem)

    @pl.loop(0, _CD)
    def _(i):
        ones[i] = ov

    @pl.loop(0, _ZCH)
    def _(i):
        zb[i] = zv

    @pl.loop(sid, _NZC, step=_NS)
    def _(c):
        pltpu.async_copy(zb, acc.at[pl.ds(c * _ZCH, _ZCH)], gsem)

    @pl.loop(sid, _NZC, step=_NS)
    def _(c):
        pltpu.make_async_copy(zb, acc.at[pl.ds(c * _ZCH, _ZCH)], gsem).wait()

    pltpu.make_async_copy(dst_hbm.at[wid], dsts, isem).wait()
    plsc.subcore_barrier()

    pltpu.async_copy(ones, acc.at[dsts.at[0]], ssem, add=True)
    pltpu.async_copy(ones, acc.at[dsts.at[1]], ssem, add=True)

    @pl.loop(0, _GD - 2)
    def _(g):
        pltpu.async_copy(ones, acc.at[dsts.at[g + 2]], ssem, add=True)
        pltpu.make_async_copy(ones, acc.at[dsts.at[g]], ssem).wait()

    pltpu.make_async_copy(ones, acc.at[dsts.at[_GD - 2]], ssem).wait()
    pltpu.make_async_copy(ones, acc.at[dsts.at[_GD - 1]], ssem).wait()
    plsc.subcore_barrier()

    @pl.loop(sid, _NZC, step=_NS)
    def _(c):
        base = c * _ZCH
        pltpu.async_copy(acc.at[pl.ds(base, _ZCH)], cnt_hbm.at[cid, pl.ds(base, _ZCH)], gsem)

    @pl.loop(sid, _NZC, step=_NS)
    def _(c):
        base = c * _ZCH
        pltpu.make_async_copy(acc.at[pl.ds(base, _ZCH)], cnt_hbm.at[cid, pl.ds(base, _ZCH)], gsem).wait()


def _spmm_body(u_hbm, ei_hbm, out_hbm, idx, rows, acc, isem, gsem, ssem):
    cid = lax.axis_index("c")
    sid = lax.axis_index("s")
    wid = cid * _NS + sid
    zv = jnp.zeros((16,), _f32)

    @pl.loop(0, _ZCH)
    def _(i):
        for j in range(_D // 16):
            rows[0, i, pl.ds(16 * j, 16)] = zv

    @pl.loop(sid, _NZC, step=_NS)
    def _(c):
        pltpu.async_copy(rows.at[0, pl.ds(0, _ZCH)], acc.at[pl.ds(c * _ZCH, _ZCH)], gsem)

    @pl.loop(sid, _NZC, step=_NS)
    def _(c):
        pltpu.make_async_copy(rows.at[0, pl.ds(0, _ZCH)], acc.at[pl.ds(c * _ZCH, _ZCH)], gsem).wait()

    plsc.subcore_barrier()

    pltpu.async_copy(ei_hbm.at[wid, 0], idx.at[0], isem)
    pltpu.make_async_copy(ei_hbm.at[wid, 0], idx.at[0], isem).wait()
    pltpu.async_copy(ei_hbm.at[wid, 1], idx.at[1], isem)
    pltpu.async_copy(u_hbm.at[idx.at[0, 0]], rows.at[0], gsem)

    @pl.loop(0, _G - 1)
    def _(g):
        rsl = lax.rem(g, 2)
        rnsl = 1 - rsl
        isl = lax.rem(g, 3)
        inx = lax.rem(g + 1, 3)
        inx2 = lax.rem(g + 2, 3)
        pltpu.make_async_copy(ei_hbm.at[wid, g + 1], idx.at[inx], isem).wait()

        @pl.when(g >= 1)
        def _():
            pltpu.make_async_copy(rows.at[rnsl], acc.at[idx.at[inx2, 1]], ssem).wait()

        pltpu.async_copy(u_hbm.at[idx.at[inx, 0]], rows.at[rnsl], gsem)
        pltpu.make_async_copy(u_hbm.at[idx.at[isl, 0]], rows.at[rsl], gsem).wait()
        pltpu.async_copy(rows.at[rsl], acc.at[idx.at[isl, 1]], ssem, add=True)

        @pl.when(g < _G - 2)
        def _():
            pltpu.async_copy(ei_hbm.at[wid, g + 2], idx.at[inx2], isem)

    lsl = (_G - 1) % 2
    lisl = (_G - 1) % 3
    pltpu.make_async_copy(rows.at[1 - lsl], acc.at[idx.at[0, 1]], ssem).wait()
    pltpu.make_async_copy(u_hbm.at[idx.at[lisl, 0]], rows.at[lsl], gsem).wait()
    pltpu.sync_copy(rows.at[lsl], acc.at[idx.at[lisl, 1]], add=True)

    plsc.subcore_barrier()

    @pl.loop(sid, _NZC, step=_NS)
    def _(c):
        base = c * _ZCH
        pltpu.async_copy(acc.at[pl.ds(base, _ZCH)], out_hbm.at[cid, pl.ds(base, _ZCH)], gsem)

    @pl.loop(sid, _NZC, step=_NS)
    def _(c):
        base = c * _ZCH
        pltpu.make_async_copy(acc.at[pl.ds(base, _ZCH)], out_hbm.at[cid, pl.ds(base, _ZCH)], gsem).wait()


@functools.cache
def _sc_kernels():
    mesh = plsc.VectorSubcoreMesh(
        core_axis_name="c", subcore_axis_name="s", num_cores=_NC, num_subcores=_NS
    )
    deg = pl.kernel(
        _deg_body,
        out_type=jax.ShapeDtypeStruct((_NC, _N, 16), _f32),
        mesh=mesh,
        scratch_types=[
            pltpu.VMEM((_GD, _CD), jnp.int32),
            pltpu.VMEM((_CD, 16), _f32),
            pltpu.VMEM((_ZCH, 16), _f32),
            pltpu.VMEM_SHARED((_N, 16), _f32),
            pltpu.SemaphoreType.DMA,
            pltpu.SemaphoreType.DMA,
            pltpu.SemaphoreType.DMA,
        ],
    )
    spmm = pl.kernel(
        _spmm_body,
        out_type=jax.ShapeDtypeStruct((_NC, _N, _D), _f32),
        mesh=mesh,
        scratch_types=[
            pltpu.VMEM((3, 2, _C), jnp.int32),
            pltpu.VMEM((2, _C, _D), _f32),
            pltpu.VMEM_SHARED((_N, _D), _f32),
            pltpu.SemaphoreType.DMA,
            pltpu.SemaphoreType.DMA,
            pltpu.SemaphoreType.DMA,
        ],
    )
    return deg, spmm


def _tc_first_body(x_ref, w_ref, cnt_ref, u_ref, dinv_ref):
    deg = cnt_ref[0, :, 0:1] + cnt_ref[1, :, 0:1] + 1.0
    dinv = lax.rsqrt(deg)
    dinv_ref[...] = dinv
    u_ref[...] = jnp.dot(x_ref[...], w_ref[...], preferred_element_type=_f32) * dinv


_tc_first = pl.pallas_call(
    _tc_first_body,
    grid=(_N // _BM,),
    in_specs=[
        pl.BlockSpec((_BM, _D), lambda i: (i, 0)),
        pl.BlockSpec((_D, _D), lambda i: (0, 0)),
        pl.BlockSpec((2, _BM, 16), lambda i: (0, i, 0)),
    ],
    out_specs=[
        pl.BlockSpec((_BM, _D), lambda i: (i, 0)),
        pl.BlockSpec((_BM, 1), lambda i: (i, 0)),
    ],
    out_shape=[
        jax.ShapeDtypeStruct((_N, _D), _f32),
        jax.ShapeDtypeStruct((_N, 1), _f32),
    ],
)


def _tc_mid_body(s_ref, u_ref, dinv_ref, b_ref, w_ref, o_ref):
    dinv = dinv_ref[...]
    agg = s_ref[0] + s_ref[1] + u_ref[...]
    h = jnp.maximum(agg * dinv + b_ref[...], 0.0)
    o_ref[...] = jnp.dot(h, w_ref[...], preferred_element_type=_f32) * dinv


_tc_mid = pl.pallas_call(
    _tc_mid_body,
    grid=(_N // _BM,),
    in_specs=[
        pl.BlockSpec((2, _BM, _D), lambda i: (0, i, 0)),
        pl.BlockSpec((_BM, _D), lambda i: (i, 0)),
        pl.BlockSpec((_BM, 1), lambda i: (i, 0)),
        pl.BlockSpec((1, _D), lambda i: (0, 0)),
        pl.BlockSpec((_D, _D), lambda i: (0, 0)),
    ],
    out_specs=pl.BlockSpec((_BM, _D), lambda i: (i, 0)),
    out_shape=jax.ShapeDtypeStruct((_N, _D), _f32),
)


def _tc_last_body(s_ref, u_ref, dinv_ref, b_ref, wl_ref, bl_ref, o_ref):
    agg = s_ref[0] + s_ref[1] + u_ref[...]
    h = jnp.maximum(agg * dinv_ref[...] + b_ref[...], 0.0)
    o_ref[...] = jnp.dot(h, wl_ref[...], preferred_element_type=_f32) + bl_ref[...]


_tc_last = pl.pallas_call(
    _tc_last_body,
    grid=(_N // _BM,),
    in_specs=[
        pl.BlockSpec((2, _BM, _D), lambda i: (0, i, 0)),
        pl.BlockSpec((_BM, _D), lambda i: (i, 0)),
        pl.BlockSpec((_BM, 1), lambda i: (i, 0)),
        pl.BlockSpec((1, _D), lambda i: (0, 0)),
        pl.BlockSpec((_D, 1), lambda i: (0, 0)),
        pl.BlockSpec((1, 1), lambda i: (0, 0)),
    ],
    out_specs=pl.BlockSpec((_BM, 1), lambda i: (i, 0)),
    out_shape=jax.ShapeDtypeStruct((_N, 1), _f32),
)


def kernel(x, edge_index, W0, b0, W1, b1, W2, b2, Wl, bl):
    _deg_kernel, _spmm = _sc_kernels()
    dst_deg = edge_index[1].reshape(_NW, _GD, _CD)
    ei_sp = edge_index.reshape(2, _NW, _G, _C).transpose(1, 2, 0, 3)
    cnt = _deg_kernel(dst_deg)
    u0, dinv = _tc_first(x, W0, cnt)
    s0 = _spmm(u0, ei_sp)
    u1 = _tc_mid(s0, u0, dinv, b0.reshape(1, _D), W1)
    s1 = _spmm(u1, ei_sp)
    u2 = _tc_mid(s1, u1, dinv, b1.reshape(1, _D), W2)
    s2 = _spmm(u2, ei_sp)
    out = _tc_last(s2, u2, dinv, b2.reshape(1, _D), Wl, bl.reshape(1, 1))
    return out[:, 0]
